# baseline (speedup 1.0000x reference)
import functools

import jax
import jax.numpy as jnp
from jax import lax
from jax.experimental import pallas as pl
from jax.experimental.pallas import tpu as pltpu
from jax.experimental.pallas import tpu_sc as plsc

B = 1024
S = 200
VOCAB = 100000
EMB = 128
NUM_LABELS = 42

NC = 2
NS = 16
NW = NC * NS
SENT_W = B // NW


def _sc_body(emb_hbm, ids_hbm, g_hbm, idx_v, rows_v,
             sg0, sg1, sg2, sg3, so0, so1, so2, so3):
    wid = lax.axis_index("s") * NC + lax.axis_index("c")
    b0 = wid * SENT_W
    pltpu.sync_copy(ids_hbm.at[pl.ds(b0 * S, SENT_W * S)], idx_v)
    sg = (sg0, sg1, sg2, sg3)
    so = (so0, so1, so2, so3)

    def start_gather(j, q):
        pltpu.async_copy(emb_hbm.at[idx_v.at[pl.ds(j * S, S)]],
                         rows_v.at[q], sg[q])

    def out_copy(j, q):
        return pltpu.make_async_copy(rows_v.at[q], g_hbm.at[:, b0 + j, :],
                                     so[q])

    start_gather(0, 0)
    start_gather(1, 1)

    def quad(g, carry):
        for q in range(4):
            j = 4 * g + q
            pltpu.make_async_copy(emb_hbm.at[idx_v.at[pl.ds(j * S, S)]],
                                  rows_v.at[q], sg[q]).wait()
            out_copy(j, q).start()
            q2 = (q + 2) % 4
            if q < 2:
                @pl.when(g > 0)
                def _():
                    out_copy(j - 2, q2).wait()

                start_gather(j + 2, q2)
            else:
                out_copy(j - 2, q2).wait()

                @pl.when(g < SENT_W // 4 - 1)
                def _():
                    start_gather(j + 2, q2)
        return carry

    lax.fori_loop(0, SENT_W // 4, quad, 0)
    for q, j in ((2, SENT_W - 2), (3, SENT_W - 1)):
        out_copy(j, q).wait()


@functools.cache
def _sc_gather():
    return pl.kernel(
        _sc_body,
        out_type=jax.ShapeDtypeStruct((S, B, EMB), jnp.float32),
        mesh=plsc.VectorSubcoreMesh(core_axis_name="c", subcore_axis_name="s",
                                    num_cores=NC, num_subcores=NS),
        scratch_types=[
            pltpu.VMEM((SENT_W * S,), jnp.int32),
            pltpu.VMEM((4, S, EMB), jnp.float32),
            pltpu.SemaphoreType.DMA,
            pltpu.SemaphoreType.DMA,
            pltpu.SemaphoreType.DMA,
            pltpu.SemaphoreType.DMA,
            pltpu.SemaphoreType.DMA,
            pltpu.SemaphoreType.DMA,
            pltpu.SemaphoreType.DMA,
            pltpu.SemaphoreType.DMA,
        ],
        compiler_params=pltpu.CompilerParams(use_tc_tiling_on_sc=False,
                                             needs_layout_passes=False),
    )


S_BLK = 8


def _cls_body(g_ref, pos_ref, wt_ref, b_ref, out_ref):
    h = (g_ref[...] + pos_ref[...][:, None, :]).reshape(S_BLK * B, EMB)
    r = lax.dot_general(wt_ref[...], h, (((1,), (1,)), ((), ())),
                        preferred_element_type=jnp.float32)
    out_ref[...] = r + b_ref[...]


def _classify(g, pos_s, w_t, b_rep):
    return pl.pallas_call(
        _cls_body,
        grid=(S // S_BLK,),
        in_specs=[
            pl.BlockSpec((S_BLK, B, EMB), lambda i: (i, 0, 0)),
            pl.BlockSpec((S_BLK, EMB), lambda i: (i, 0)),
            pl.BlockSpec((NUM_LABELS, EMB), lambda i: (0, 0)),
            pl.BlockSpec((NUM_LABELS, S_BLK * B), lambda i: (0, 0)),
        ],
        out_specs=pl.BlockSpec((NUM_LABELS, S_BLK * B), lambda i: (0, i)),
        out_shape=jax.ShapeDtypeStruct((NUM_LABELS, S * B), jnp.float32),
    )(g, pos_s, w_t, b_rep)


def kernel(input_ids, emb_table, pos_table, W_cls, b_cls):
    ids_flat = input_ids.reshape(-1).astype(jnp.int32)
    g = _sc_gather()(emb_table, ids_flat)
    w_t = W_cls.T
    b_rep = jnp.broadcast_to(b_cls[:, None], (NUM_LABELS, S_BLK * B))
    out_t = _classify(g, pos_table[:S], w_t, b_rep)
    return jnp.transpose(out_t.reshape(NUM_LABELS, S, B), (2, 1, 0))


# speedup vs baseline: 1.4024x; 1.4024x over previous
import functools

import jax
import jax.numpy as jnp
from jax import lax
from jax.experimental import pallas as pl
from jax.experimental.pallas import tpu as pltpu
from jax.experimental.pallas import tpu_sc as plsc

B = 1024
S = 200
VOCAB = 100000
EMB = 128
NUM_LABELS = 42

NC = 2
NS = 16
NW = NC * NS
SENT_W = B // NW


def _sc_body(emb_hbm, ids_hbm, g_hbm, idx_v, rows_v,
             sg0, sg1, sg2, sg3, so0, so1, so2, so3):
    wid = lax.axis_index("s") * NC + lax.axis_index("c")
    b0 = wid * SENT_W
    pltpu.sync_copy(ids_hbm.at[pl.ds(b0 * S, SENT_W * S)], idx_v)
    sg = (sg0, sg1, sg2, sg3)
    so = (so0, so1, so2, so3)

    def start_gather(j, q):
        pltpu.async_copy(emb_hbm.at[idx_v.at[pl.ds(j * S, S)]],
                         rows_v.at[q], sg[q])

    def out_copy(j, q):
        return pltpu.make_async_copy(rows_v.at[q], g_hbm.at[:, b0 + j, :],
                                     so[q])

    start_gather(0, 0)
    start_gather(1, 1)

    def quad(g, carry):
        for q in range(4):
            j = 4 * g + q
            pltpu.make_async_copy(emb_hbm.at[idx_v.at[pl.ds(j * S, S)]],
                                  rows_v.at[q], sg[q]).wait()
            out_copy(j, q).start()
            q2 = (q + 2) % 4
            if q < 2:
                @pl.when(g > 0)
                def _():
                    out_copy(j - 2, q2).wait()

                start_gather(j + 2, q2)
            else:
                out_copy(j - 2, q2).wait()

                @pl.when(g < SENT_W // 4 - 1)
                def _():
                    start_gather(j + 2, q2)
        return carry

    lax.fori_loop(0, SENT_W // 4, quad, 0)
    for q, j in ((2, SENT_W - 2), (3, SENT_W - 1)):
        out_copy(j, q).wait()


@functools.cache
def _sc_gather():
    return pl.kernel(
        _sc_body,
        out_type=jax.ShapeDtypeStruct((S, B, EMB), jnp.float32),
        mesh=plsc.VectorSubcoreMesh(core_axis_name="c", subcore_axis_name="s",
                                    num_cores=NC, num_subcores=NS),
        scratch_types=[
            pltpu.VMEM((SENT_W * S,), jnp.int32),
            pltpu.VMEM((4, S, EMB), jnp.float32),
            pltpu.SemaphoreType.DMA,
            pltpu.SemaphoreType.DMA,
            pltpu.SemaphoreType.DMA,
            pltpu.SemaphoreType.DMA,
            pltpu.SemaphoreType.DMA,
            pltpu.SemaphoreType.DMA,
            pltpu.SemaphoreType.DMA,
            pltpu.SemaphoreType.DMA,
        ],
        compiler_params=pltpu.CompilerParams(use_tc_tiling_on_sc=False,
                                             needs_layout_passes=False),
    )


S_BLK = 40


def _cls_body(g_ref, pos_ref, wt_ref, b_ref, out_ref):
    h = g_ref[...] + pos_ref[...][:, None, :]
    for s in range(S_BLK):
        r = lax.dot_general(wt_ref[...], h[s], (((1,), (1,)), ((), ())),
                            preferred_element_type=jnp.float32)
        out_ref[:, s, :] = r + b_ref[...]


def _classify(g, pos_s, w_t, b_rep):
    return pl.pallas_call(
        _cls_body,
        grid=(S // S_BLK,),
        in_specs=[
            pl.BlockSpec((S_BLK, B, EMB), lambda i: (i, 0, 0)),
            pl.BlockSpec((S_BLK, EMB), lambda i: (i, 0)),
            pl.BlockSpec((NUM_LABELS, EMB), lambda i: (0, 0)),
            pl.BlockSpec((NUM_LABELS, B), lambda i: (0, 0)),
        ],
        out_specs=pl.BlockSpec((NUM_LABELS, S_BLK, B), lambda i: (0, i, 0)),
        out_shape=jax.ShapeDtypeStruct((NUM_LABELS, S, B), jnp.float32),
    )(g, pos_s, w_t, b_rep)


def kernel(input_ids, emb_table, pos_table, W_cls, b_cls):
    ids_flat = input_ids.reshape(-1).astype(jnp.int32)
    g = _sc_gather()(emb_table, ids_flat)
    w_t = W_cls.T
    b_rep = jnp.broadcast_to(b_cls[:, None], (NUM_LABELS, B))
    out_t = _classify(g, pos_table[:S], w_t, b_rep)
    return jnp.transpose(out_t, (2, 1, 0))


# speedup vs baseline: 1.4126x; 1.0073x over previous
import functools

import jax
import jax.numpy as jnp
from jax import lax
from jax.experimental import pallas as pl
from jax.experimental.pallas import tpu as pltpu
from jax.experimental.pallas import tpu_sc as plsc

B = 1024
S = 200
VOCAB = 100000
EMB = 128
NUM_LABELS = 42

NC = 2
NS = 16
NW = NC * NS
SENT_W = B // NW

S_SPLIT = 120
S_BLK = 40


def _sc_body(s_off, s_len, emb_hbm, ids_hbm, g_hbm, idx_v, rows_v,
             sg0, sg1, sg2, sg3, so0, so1, so2, so3):
    wid = lax.axis_index("s") * NC + lax.axis_index("c")
    b0 = wid * SENT_W
    pltpu.sync_copy(ids_hbm.at[pl.ds(b0 * S, SENT_W * S)], idx_v)
    sg = (sg0, sg1, sg2, sg3)
    so = (so0, so1, so2, so3)

    def start_gather(j, q):
        pltpu.async_copy(emb_hbm.at[idx_v.at[pl.ds(j * S + s_off, s_len)]],
                         rows_v.at[q], sg[q])

    def out_copy(j, q):
        return pltpu.make_async_copy(rows_v.at[q], g_hbm.at[:, b0 + j, :],
                                     so[q])

    start_gather(0, 0)
    start_gather(1, 1)

    def quad(g, carry):
        for q in range(4):
            j = 4 * g + q
            pltpu.make_async_copy(
                emb_hbm.at[idx_v.at[pl.ds(j * S + s_off, s_len)]],
                rows_v.at[q], sg[q]).wait()
            out_copy(j, q).start()
            q2 = (q + 2) % 4
            if q < 2:
                @pl.when(g > 0)
                def _():
                    out_copy(j - 2, q2).wait()

                start_gather(j + 2, q2)
            else:
                out_copy(j - 2, q2).wait()

                @pl.when(g < SENT_W // 4 - 1)
                def _():
                    start_gather(j + 2, q2)
        return carry

    lax.fori_loop(0, SENT_W // 4, quad, 0)
    for q, j in ((2, SENT_W - 2), (3, SENT_W - 1)):
        out_copy(j, q).wait()


@functools.cache
def _sc_gather(s_off, s_len):
    return pl.kernel(
        functools.partial(_sc_body, s_off, s_len),
        out_type=jax.ShapeDtypeStruct((s_len, B, EMB), jnp.float32),
        mesh=plsc.VectorSubcoreMesh(core_axis_name="c", subcore_axis_name="s",
                                    num_cores=NC, num_subcores=NS),
        scratch_types=[
            pltpu.VMEM((SENT_W * S,), jnp.int32),
            pltpu.VMEM((4, s_len, EMB), jnp.float32),
            pltpu.SemaphoreType.DMA,
            pltpu.SemaphoreType.DMA,
            pltpu.SemaphoreType.DMA,
            pltpu.SemaphoreType.DMA,
            pltpu.SemaphoreType.DMA,
            pltpu.SemaphoreType.DMA,
            pltpu.SemaphoreType.DMA,
            pltpu.SemaphoreType.DMA,
        ],
        compiler_params=pltpu.CompilerParams(use_tc_tiling_on_sc=False,
                                             needs_layout_passes=False),
    )


def _cls_body(g_ref, pos_ref, wt_ref, b_ref, out_ref):
    h = g_ref[...] + pos_ref[...][:, None, :]
    for s in range(S_BLK):
        r = lax.dot_general(wt_ref[...], h[s], (((1,), (1,)), ((), ())),
                            preferred_element_type=jnp.float32)
        out_ref[:, s, :] = r + b_ref[...]


def _cls_body_acc(prev_ref, g_ref, pos_ref, wt_ref, b_ref, out_ref):
    del prev_ref
    _cls_body(g_ref, pos_ref, wt_ref, b_ref, out_ref)


def _classify_first(g, pos_s, w_t, b_rep):
    return pl.pallas_call(
        _cls_body,
        grid=(S_SPLIT // S_BLK,),
        in_specs=[
            pl.BlockSpec((S_BLK, B, EMB), lambda i: (i, 0, 0)),
            pl.BlockSpec((S_BLK, EMB), lambda i: (i, 0)),
            pl.BlockSpec((NUM_LABELS, EMB), lambda i: (0, 0)),
            pl.BlockSpec((NUM_LABELS, B), lambda i: (0, 0)),
        ],
        out_specs=pl.BlockSpec((NUM_LABELS, S_BLK, B), lambda i: (0, i, 0)),
        out_shape=jax.ShapeDtypeStruct((NUM_LABELS, S, B), jnp.float32),
    )(g, pos_s, w_t, b_rep)


def _classify_rest(prev, g, pos_s, w_t, b_rep):
    blk0 = S_SPLIT // S_BLK
    return pl.pallas_call(
        _cls_body_acc,
        grid=((S - S_SPLIT) // S_BLK,),
        in_specs=[
            pl.BlockSpec(memory_space=pl.ANY),
            pl.BlockSpec((S_BLK, B, EMB), lambda i: (i, 0, 0)),
            pl.BlockSpec((S_BLK, EMB), lambda i: (i, 0)),
            pl.BlockSpec((NUM_LABELS, EMB), lambda i: (0, 0)),
            pl.BlockSpec((NUM_LABELS, B), lambda i: (0, 0)),
        ],
        out_specs=pl.BlockSpec((NUM_LABELS, S_BLK, B),
                               lambda i: (0, i + blk0, 0)),
        out_shape=jax.ShapeDtypeStruct((NUM_LABELS, S, B), jnp.float32),
        input_output_aliases={0: 0},
    )(prev, g, pos_s, w_t, b_rep)


def kernel(input_ids, emb_table, pos_table, W_cls, b_cls):
    ids_flat = input_ids.reshape(-1).astype(jnp.int32)
    w_t = W_cls.T
    b_rep = jnp.broadcast_to(b_cls[:, None], (NUM_LABELS, B))
    g1 = _sc_gather(0, S_SPLIT)(emb_table, ids_flat)
    g2 = _sc_gather(S_SPLIT, S - S_SPLIT)(emb_table, ids_flat)
    out1 = _classify_first(g1, pos_table[:S_SPLIT], w_t, b_rep)
    out_t = _classify_rest(out1, g2, pos_table[S_SPLIT:S], w_t, b_rep)
    return jnp.transpose(out_t, (2, 1, 0))
